# baseline (speedup 1.0000x reference)
import functools

import jax
import jax.numpy as jnp
from jax import lax
from jax.experimental import pallas as pl
from jax.experimental.pallas import tpu as pltpu
from jax.experimental.pallas import tpu_sc as plsc

EPS = 1e-5
F32 = jnp.float32

NC, NSUB = 2, 16
NW = NC * NSUB

B_IMG = 16
GRID = 66
ROWS = GRID * GRID
PADR = 67
XROWS = ROWS + 2 * PADR
TAB_PIX = B_IMG * ROWS

NP0 = 50000
NPP = 51200
NS0 = 10000
NSP = 10240
EP0, EPP = 800000, 802816
ES0, ESP = 320000, 327680
CHK = 128
CNT_N = 73728
CNT_E = 1212416


def _sc_mesh():
  return plsc.VectorSubcoreMesh(core_axis_name="c", subcore_axis_name="s")


_SC_PARAMS = pltpu.CompilerParams(use_tc_tiling_on_sc=False)


def _pix_gather(table, idx2d):
  CH, CBATCH = 64, 5
  epw = NPP // NW
  nb = epw // (CH * CBATCH)

  @functools.partial(
      pl.kernel,
      out_type=jax.ShapeDtypeStruct((NPP, 64), F32),
      mesh=_sc_mesh(),
      compiler_params=_SC_PARAMS,
      scratch_types=[
          pltpu.VMEM((CBATCH, CH), jnp.int32),
          pltpu.VMEM((CBATCH, CH, 64), F32),
          pltpu.SemaphoreType.DMA,
      ],
  )
  def k(tab_hbm, idx_hbm, out_hbm, idx_v, rows_v, sem):
    cid = lax.axis_index("c")
    sid = lax.axis_index("s")
    wid = sid * NC + cid
    base_chunk = wid * (epw // CH)

    def body(b, _):
      row0 = base_chunk + b * CBATCH
      pltpu.sync_copy(idx_hbm.at[wid * nb + b], idx_v)
      descs = [
          pltpu.async_copy(tab_hbm.at[idx_v.at[j]], rows_v.at[j], sem)
          for j in range(CBATCH)
      ]
      for j in range(CBATCH):
        descs[j].wait()
        pltpu.sync_copy(rows_v.at[j],
                        out_hbm.at[pl.ds((row0 + j) * CH, CH)])
      return 0

    lax.fori_loop(0, nb, body, 0)

  return k(table, idx2d)


def _edge_agg(table_flat, srcoff, dst2d, n_out, feat, nq, e_pad, cbatch, chk=CHK, dr=128):
  nqc = nq // NC
  ept = e_pad // NSUB
  nb = ept // (chk * cbatch)
  rpt = n_out // NSUB
  DR = dr
  nzc = rpt // DR

  @functools.partial(
      pl.kernel,
      out_type=jax.ShapeDtypeStruct((nq * n_out, feat), F32),
      mesh=_sc_mesh(),
      compiler_params=_SC_PARAMS,
      scratch_types=[
          pltpu.VMEM((2, cbatch, chk), jnp.int32),
          pltpu.VMEM((2, cbatch, chk), jnp.int32),
          pltpu.VMEM((2, cbatch, chk, feat), F32),
          pltpu.VMEM((DR, feat), F32),
          pltpu.VMEM((DR, feat), F32),
          pltpu.VMEM_SHARED((n_out, feat), F32),
          pltpu.SemaphoreType.DMA,
          pltpu.SemaphoreType.DMA,
          pltpu.SemaphoreType.DMA,
      ],
  )
  def k(tab_hbm, src_hbm, dst_hbm, zeros_hbm, out_hbm,
        sidx, didx, rows, zb, dump, acc, sem, isem, ssem):
    cid = lax.axis_index("c")
    sid = lax.axis_index("s")
    pltpu.sync_copy(zeros_hbm, zb)
    base_b = sid * nb

    for qq in range(nqc):
      q = cid * nqc + qq

      def zbody(i, _):
        pltpu.sync_copy(zb, acc.at[pl.ds(sid * rpt + i * DR, DR)])
        return 0
      lax.fori_loop(0, nzc, zbody, 0)
      plsc.subcore_barrier()

      pltpu.sync_copy(src_hbm.at[q, base_b], sidx.at[0])
      pltpu.sync_copy(dst_hbm.at[base_b], didx.at[0])

      def pbody(b2, _):
        e0 = base_b + 2 * b2
        g0 = [
            pltpu.async_copy(tab_hbm.at[sidx.at[0, j]], rows.at[0, j], sem)
            for j in range(cbatch)
        ]
        pltpu.async_copy(src_hbm.at[q, e0 + 1], sidx.at[1], isem)
        pltpu.async_copy(dst_hbm.at[e0 + 1], didx.at[1], isem)
        s0 = []
        for j in range(cbatch):
          g0[j].wait()
          s0.append(pltpu.async_copy(rows.at[0, j], acc.at[didx.at[0, j]],
                                     ssem, add=True))
        pltpu.make_async_copy(src_hbm.at[q, e0], sidx.at[1], isem).wait()
        pltpu.make_async_copy(dst_hbm.at[e0], didx.at[1], isem).wait()
        g1 = [
            pltpu.async_copy(tab_hbm.at[sidx.at[1, j]], rows.at[1, j], sem)
            for j in range(cbatch)
        ]
        for d in s0:
          d.wait()

        @pl.when(b2 + 1 < nb // 2)
        def _():
          pltpu.async_copy(src_hbm.at[q, e0 + 2], sidx.at[0], isem)
          pltpu.async_copy(dst_hbm.at[e0 + 2], didx.at[0], isem)

        s1 = []
        for j in range(cbatch):
          g1[j].wait()
          s1.append(pltpu.async_copy(rows.at[1, j], acc.at[didx.at[1, j]],
                                     ssem, add=True))
        for d in s1:
          d.wait()

        @pl.when(b2 + 1 < nb // 2)
        def _():
          pltpu.make_async_copy(src_hbm.at[q, e0], sidx.at[0], isem).wait()
          pltpu.make_async_copy(dst_hbm.at[e0], didx.at[0], isem).wait()
        return 0
      lax.fori_loop(0, nb // 2, pbody, 0)
      plsc.subcore_barrier()

      def dbody(i, _):
        r0 = sid * rpt + i * DR
        pltpu.sync_copy(acc.at[pl.ds(r0, DR)], dump)
        pltpu.sync_copy(dump, out_hbm.at[pl.ds(q * n_out + r0, DR)])
        return 0
      lax.fori_loop(0, nzc, dbody, 0)
      plsc.subcore_barrier()

  return k(table_flat, srcoff, dst2d, jnp.zeros((DR, feat), F32))


def _edge_agg_full(table, srcoff, dst2d, n_out, e_pad, cbatch, chk, dr=32):
  nb = e_pad // (NW * chk * cbatch)
  rpt = n_out // NSUB
  DR = dr
  nzc = rpt // DR

  @functools.partial(
      pl.kernel,
      out_type=jax.ShapeDtypeStruct((2 * n_out, 128), F32),
      mesh=_sc_mesh(),
      scratch_types=[
          pltpu.VMEM((2, cbatch, chk), jnp.int32),
          pltpu.VMEM((2, cbatch, chk), jnp.int32),
          pltpu.VMEM((2, cbatch, chk, 128), F32),
          pltpu.VMEM((DR, 128), F32),
          pltpu.VMEM((DR, 128), F32),
          pltpu.VMEM_SHARED((n_out, 128), F32),
          pltpu.SemaphoreType.DMA,
          pltpu.SemaphoreType.DMA,
          pltpu.SemaphoreType.DMA,
      ],
  )
  def k(tab_hbm, src_hbm, dst_hbm, zeros_hbm, out_hbm,
        sidx, didx, rows, zb, dump, acc, sem, isem, ssem):
    cid = lax.axis_index("c")
    sid = lax.axis_index("s")
    wid = sid * NC + cid
    pltpu.sync_copy(zeros_hbm, zb)
    base_b = wid * nb

    def zbody(i, _):
      pltpu.sync_copy(zb, acc.at[pl.ds(sid * rpt + i * DR, DR)])
      return 0
    lax.fori_loop(0, nzc, zbody, 0)
    plsc.subcore_barrier()

    pltpu.sync_copy(src_hbm.at[0, base_b], sidx.at[0])
    pltpu.sync_copy(dst_hbm.at[base_b], didx.at[0])

    def pbody(b2, _):
      e0 = base_b + 2 * b2
      g0 = [
          pltpu.async_copy(tab_hbm.at[sidx.at[0, j]], rows.at[0, j], sem)
          for j in range(cbatch)
      ]
      pltpu.async_copy(src_hbm.at[0, e0 + 1], sidx.at[1], isem)
      pltpu.async_copy(dst_hbm.at[e0 + 1], didx.at[1], isem)
      s0 = []
      for j in range(cbatch):
        g0[j].wait()
        s0.append(pltpu.async_copy(rows.at[0, j], acc.at[didx.at[0, j]],
                                   ssem, add=True))
      pltpu.make_async_copy(src_hbm.at[0, e0], sidx.at[1], isem).wait()
      pltpu.make_async_copy(dst_hbm.at[e0], didx.at[1], isem).wait()
      g1 = [
          pltpu.async_copy(tab_hbm.at[sidx.at[1, j]], rows.at[1, j], sem)
          for j in range(cbatch)
      ]
      for d in s0:
        d.wait()

      @pl.when(b2 + 1 < nb // 2)
      def _():
        pltpu.async_copy(src_hbm.at[0, e0 + 2], sidx.at[0], isem)
        pltpu.async_copy(dst_hbm.at[e0 + 2], didx.at[0], isem)

      s1 = []
      for j in range(cbatch):
        g1[j].wait()
        s1.append(pltpu.async_copy(rows.at[1, j], acc.at[didx.at[1, j]],
                                   ssem, add=True))
      for d in s1:
        d.wait()

      @pl.when(b2 + 1 < nb // 2)
      def _():
        pltpu.make_async_copy(src_hbm.at[0, e0], sidx.at[0], isem).wait()
        pltpu.make_async_copy(dst_hbm.at[e0], didx.at[0], isem).wait()
      return 0
    lax.fori_loop(0, nb // 2, pbody, 0)
    plsc.subcore_barrier()

    def dbody(i, _):
      r0 = sid * rpt + i * DR
      pltpu.sync_copy(acc.at[pl.ds(r0, DR)], dump)
      pltpu.sync_copy(dump, out_hbm.at[pl.ds(cid * n_out + r0, DR)])
      return 0
    lax.fori_loop(0, nzc, dbody, 0)

  return k(table, srcoff, dst2d, jnp.zeros((DR, 128), F32))


def _counts(dst2d, ones_h, zeros_h):
  CBATCH = 8
  epw = CNT_E // NW
  nb = epw // (CHK * CBATCH)
  rpt = CNT_N // NSUB
  DR = 512
  nzc = rpt // DR

  @functools.partial(
      pl.kernel,
      out_type=jax.ShapeDtypeStruct((2, CNT_N), F32),
      mesh=_sc_mesh(),
      compiler_params=_SC_PARAMS,
      scratch_types=[
          pltpu.VMEM((CBATCH, CHK), jnp.int32),
          pltpu.VMEM((CHK,), F32),
          pltpu.VMEM((DR,), F32),
          pltpu.VMEM((DR,), F32),
          pltpu.VMEM_SHARED((CNT_N,), F32),
      ],
  )
  def k(dst_hbm, ones_hbm, zeros_hbm, out_hbm, didx, ones_v, zb, dump, acc):
    cid = lax.axis_index("c")
    sid = lax.axis_index("s")
    wid = sid * NC + cid
    pltpu.sync_copy(ones_hbm, ones_v)
    pltpu.sync_copy(zeros_hbm, zb)

    def zbody(i, _):
      pltpu.sync_copy(zb, acc.at[pl.ds(sid * rpt + i * DR, DR)])
      return 0
    lax.fori_loop(0, nzc, zbody, 0)
    plsc.subcore_barrier()

    def ebody(b, _):
      pltpu.sync_copy(dst_hbm.at[wid * nb + b], didx)
      for j in range(CBATCH):
        pltpu.sync_copy(ones_v, acc.at[didx.at[j]], add=True)
      return 0
    lax.fori_loop(0, nb, ebody, 0)
    plsc.subcore_barrier()

    def dbody(i, _):
      r0 = sid * rpt + i * DR
      pltpu.sync_copy(acc.at[pl.ds(r0, DR)], dump)
      pltpu.sync_copy(dump, out_hbm.at[cid, pl.ds(r0, DR)])
      return 0
    lax.fori_loop(0, nzc, dbody, 0)

  return k(dst2d, ones_h, zeros_h)


def _conv_frontend(xpad, w1k, s1, t1, w2k, s2, t2, mask):
  def body(x_ref, w1_ref, s1_ref, t1_ref, w2_ref, s2_ref, t2_ref, m_ref,
           out_ref):
    x = x_ref[0]
    acc = jnp.zeros((ROWS, 64), F32)
    for k in range(9):
      o = PADR + (k // 3 - 1) * GRID + (k % 3 - 1)
      acc = acc + jnp.dot(x[o:o + ROWS, :], w1_ref[k],
                          preferred_element_type=F32)
    h = jnp.maximum(acc * s1_ref[...] + t1_ref[...], 0.0) * m_ref[...]
    zer = jnp.zeros((PADR, 64), F32)
    xp = jnp.concatenate([zer, h, zer], axis=0)
    acc2 = jnp.zeros((ROWS, 64), F32)
    for k in range(9):
      o = PADR + (k // 3 - 1) * GRID + (k % 3 - 1)
      acc2 = acc2 + jnp.dot(xp[o:o + ROWS, :], w2_ref[k],
                            preferred_element_type=F32)
    out_ref[0] = jnp.maximum(acc2 * s2_ref[...] + t2_ref[...], 0.0)

  def full(shp):
    return pl.BlockSpec(shp, lambda b: (0,) * len(shp))

  return pl.pallas_call(
      body,
      out_shape=jax.ShapeDtypeStruct((B_IMG, ROWS, 64), F32),
      grid=(B_IMG,),
      in_specs=[
          pl.BlockSpec((1, XROWS, 64), lambda b: (b, 0, 0)),
          full((9, 64, 64)), full((1, 64)), full((1, 64)),
          full((9, 64, 64)), full((1, 64)), full((1, 64)),
          full((ROWS, 1)),
      ],
      out_specs=pl.BlockSpec((1, ROWS, 64), lambda b: (b, 0, 0)),
  )(xpad, w1k, s1, t1, w2k, s2, t2, mask)


def _linear(x_parts, w, s, t, counts, n, blk, sin, win, sout, relu):
  fout = w.shape[1]
  wout = fout // sout
  use_cnt = counts is not None

  def body(*refs):
    if use_cnt:
      x_ref, w_ref, s_ref, t_ref, c_ref, out_ref = refs
    else:
      x_ref, w_ref, s_ref, t_ref, out_ref = refs
    y = jnp.zeros((blk, fout), F32)
    for p in range(sin):
      y = y + jnp.dot(x_ref[p], w_ref[p * win:(p + 1) * win, :],
                      preferred_element_type=F32)
    if use_cnt:
      c = jnp.maximum(c_ref[0] + c_ref[1], 1.0)
      y = y / c
    y = y * s_ref[...] + t_ref[...]
    if relu:
      y = jnp.maximum(y, 0.0)
    for q in range(sout):
      out_ref[q] = y[:, q * wout:(q + 1) * wout]

  in_specs = [
      pl.BlockSpec((sin, blk, win), lambda i: (0, i, 0)),
      pl.BlockSpec((sin * win, fout), lambda i: (0, 0)),
      pl.BlockSpec((1, fout), lambda i: (0, 0)),
      pl.BlockSpec((1, fout), lambda i: (0, 0)),
  ]
  args = [x_parts, w, s, t]
  if use_cnt:
    in_specs.append(pl.BlockSpec((2, blk, 1), lambda i: (0, i, 0)))
    args.append(counts)
  return pl.pallas_call(
      body,
      out_shape=jax.ShapeDtypeStruct((sout, n, wout), F32),
      grid=(n // blk,),
      in_specs=in_specs,
      out_specs=pl.BlockSpec((sout, blk, wout), lambda i: (0, i, 0)),
  )(*args)


def _readout(g_parts, ids, att_w, ro_w):
  blk = 2048
  nsteps = NSP // blk
  neg = float('-inf')

  def body(g_ref, id_ref, aw_ref, rw_ref, out_ref, hg):
    i = pl.program_id(0)

    @pl.when(i == 0)
    def _():
      hg[...] = jnp.full((16, 128), neg, F32)

    gg = g_ref[0]
    att = jnp.dot(gg, aw_ref[...], preferred_element_type=F32)
    sc = (1.0 / (1.0 + jnp.exp(-att)) + 1.0) * 0.5
    xx = gg * sc
    ids_b = id_ref[...]
    for gph in range(16):
      m = ids_b == gph
      c = jnp.max(jnp.where(m, xx, neg), axis=0, keepdims=True)
      hg[gph:gph + 1, :] = jnp.maximum(hg[gph:gph + 1, :], c)

    @pl.when(i == nsteps - 1)
    def _():
      out_ref[...] = jnp.dot(hg[...], rw_ref[...],
                             preferred_element_type=F32)

  return pl.pallas_call(
      body,
      out_shape=jax.ShapeDtypeStruct((16, 10), F32),
      grid=(nsteps,),
      in_specs=[
          pl.BlockSpec((1, blk, 128), lambda i: (0, i, 0)),
          pl.BlockSpec((blk, 1), lambda i: (i, 0)),
          pl.BlockSpec((128, 1), lambda i: (0, 0)),
          pl.BlockSpec((128, 10), lambda i: (0, 0)),
      ],
      out_specs=pl.BlockSpec((16, 10), lambda i: (0, 0)),
      scratch_shapes=[pltpu.VMEM((16, 128), F32)],
  )(g_parts, ids, att_w, ro_w)


def _fold_bn(g, b):
  s = g / jnp.sqrt(1.0 + EPS)
  return s[None, :], b[None, :]


def _fold_conv_bn(cb, g, b):
  s = g / jnp.sqrt(1.0 + EPS)
  return s[None, :], (cb * s + b)[None, :]


def _conv_w(w):
  wk = jnp.transpose(w, (2, 3, 1, 0)).reshape(9, w.shape[1], w.shape[0])
  if wk.shape[1] < 64:
    wk = jnp.pad(wk, ((0, 0), (0, 64 - wk.shape[1]), (0, 0)))
  return wk


def _pad_edges(src, dst, e_pad, n_tab, n_out, nq, cbatch, chk=CHK):
  e = src.shape[0]
  pad = e_pad - e
  ar = jnp.arange(pad, dtype=jnp.int32)
  src_p = jnp.concatenate([src.astype(jnp.int32), ar % 16])
  dst_p = jnp.concatenate([dst.astype(jnp.int32), n_out + (ar % 16)])
  offs = (jnp.arange(nq, dtype=jnp.int32) * n_tab)[:, None]
  nbatch = e_pad // (chk * cbatch)
  srcoff = (src_p[None, :] + offs).reshape(nq, nbatch, cbatch, chk)
  return srcoff, dst_p.reshape(nbatch, cbatch, chk)


def kernel(images, data_where, pixel_edge_index, pixel_batch, edge_index,
           graph_batch, params):
  p = params

  x = jnp.transpose(images, (0, 2, 3, 1))
  x = jnp.pad(x, ((0, 0), (1, 1), (1, 1), (0, 61)))
  x = x.reshape(B_IMG, ROWS, 64)
  x = jnp.pad(x, ((0, 0), (PADR, PADR), (0, 0)))

  rr = jnp.arange(ROWS)
  hh, ww = rr // GRID, rr % GRID
  interior = (hh >= 1) & (hh <= 64) & (ww >= 1) & (ww <= 64)
  mask = interior.astype(F32)[:, None]

  s1c, t1c = _fold_conv_bn(p['conv1_b'], p['bn1_g'], p['bn1_b'])
  s2c, t2c = _fold_conv_bn(p['conv2_b'], p['bn2_g'], p['bn2_b'])
  feat = _conv_frontend(x, _conv_w(p['conv1_w']), s1c, t1c,
                        _conv_w(p['conv2_w']), s2c, t2c, mask)
  table0 = feat.reshape(TAB_PIX, 64)

  dw = data_where.astype(jnp.int32)
  pidx = dw[:, 0] * ROWS + (dw[:, 1] + 1) * GRID + (dw[:, 2] + 1)
  pidx = jnp.concatenate(
      [pidx, jnp.zeros((NPP - NP0,), jnp.int32)]).reshape(160, 5, 64)
  pix = _pix_gather(table0, pidx)

  src_px = pixel_edge_index[0].astype(jnp.int32)
  dst_px = pixel_edge_index[1].astype(jnp.int32)
  src_sp = edge_index[0].astype(jnp.int32)
  dst_sp = edge_index[1].astype(jnp.int32)
  pb = pixel_batch.astype(jnp.int32)

  cnt_dummy = 71680
  parts = [dst_px, jnp.full((EPP - EP0,), cnt_dummy, jnp.int32),
           NPP + pb, jnp.full((NPP - NP0,), cnt_dummy, jnp.int32),
           NPP + NSP + dst_sp, jnp.full((ESP - ES0,), cnt_dummy, jnp.int32)]
  all_dst = jnp.concatenate(parts)
  all_dst = jnp.concatenate(
      [all_dst, jnp.full((CNT_E - all_dst.shape[0],), cnt_dummy, jnp.int32)])
  counts2 = _counts(all_dst.reshape(CNT_E // (CHK * 8), 8, CHK),
                    jnp.ones((CHK,), F32), jnp.zeros((512,), F32))
  cnt_px = counts2[:, :NPP].reshape(2, NPP, 1)
  cnt_pool = counts2[:, NPP:NPP + NSP].reshape(2, NSP, 1)
  cnt_sp = counts2[:, NPP + NSP:NPP + 2 * NSP].reshape(2, NSP, 1)

  s_emb, t_emb = jnp.ones((1, 64), F32), p['s1_emb_b'][None, :]
  h0 = _linear(pix.reshape(1, NPP, 64), p['s1_emb_w'], s_emb, t_emb,
               None, NPP, 3200, 1, 64, 2, False)

  so_px2, dst2_px = _pad_edges(src_px, dst_px, EPP, NPP, NP0, 2, 2, 128)
  agg0 = _edge_agg(h0.reshape(2 * NPP, 32), so_px2, dst2_px,
                   NPP, 32, 2, EPP, 2, 128, 64)
  bs, bt = _fold_bn(p['s1_bn0_g'], p['s1_bn0_b'])
  bt = p['s1_l0_b'][None, :] * bs + bt
  h1 = _linear(agg0.reshape(2, NPP, 32), p['s1_l0_w'], bs, bt,
               cnt_px, NPP, 3200, 2, 32, 4, True)

  so_px4, _ = _pad_edges(src_px, dst_px, EPP, NPP, NP0, 4, 2, 128)
  agg1 = _edge_agg(h1.reshape(4 * NPP, 32), so_px4, dst2_px,
                   NPP, 32, 4, EPP, 2, 128, 64)
  bs, bt = _fold_bn(p['s1_bn1_g'], p['s1_bn1_b'])
  bt = p['s1_l1_b'][None, :] * bs + bt
  h2 = _linear(agg1.reshape(4, NPP, 32), p['s1_l1_w'], bs, bt,
               cnt_px, NPP, 3200, 4, 32, 1, True)

  pool_src = jnp.arange(NPP, dtype=jnp.int32)
  pool_dst = jnp.concatenate(
      [pb, NS0 + (jnp.arange(NPP - NP0, dtype=jnp.int32) % 16)])
  so_pool, dst2_pool = _pad_edges(pool_src, pool_dst, NPP, NPP, NS0, 1, 2, 50)
  gcn1 = _edge_agg_full(h2.reshape(NPP, 128), so_pool, dst2_pool,
                        NSP, NPP, 2, 50)

  s_emb2, t_emb2 = jnp.ones((1, 128), F32), p['s2_emb_b'][None, :]
  w_emb2 = jnp.concatenate([p['s2_emb_w'], p['s2_emb_w']], axis=0)
  g = _linear(gcn1.reshape(2, NSP, 128), w_emb2, s_emb2, t_emb2,
              cnt_pool, NSP, 2048, 2, 128, 1, False)

  so_sp, dst2_sp = _pad_edges(src_sp, dst_sp, ESP, NSP, NS0, 1, 2, 64)
  for i in range(4):
    agg = _edge_agg_full(g.reshape(NSP, 128), so_sp, dst2_sp,
                         NSP, ESP, 2, 64)
    bs, bt = _fold_bn(p['s2_bn%d_g' % i], p['s2_bn%d_b' % i])
    bt = p['s2_l%d_b' % i][None, :] * bs + bt
    wl = jnp.concatenate([p['s2_l%d_w' % i], p['s2_l%d_w' % i]], axis=0)
    g = _linear(agg.reshape(2, NSP, 128), wl, bs, bt,
                cnt_sp, NSP, 2048, 2, 128, 1, True)

  gb = jnp.concatenate(
      [graph_batch.astype(jnp.int32),
       jnp.full((NSP - NS0,), 16, jnp.int32)]).reshape(NSP, 1)
  return _readout(g.reshape(1, NSP, 128), gb, p['att_w'], p['ro_w'])


# speedup vs baseline: 1.0796x; 1.0796x over previous
import functools

import jax
import jax.numpy as jnp
from jax import lax
from jax.experimental import pallas as pl
from jax.experimental.pallas import tpu as pltpu
from jax.experimental.pallas import tpu_sc as plsc

EPS = 1e-5
F32 = jnp.float32

NC, NSUB = 2, 16
NW = NC * NSUB

B_IMG = 16
GRID = 66
ROWS = GRID * GRID
PADR = 67
XROWS = ROWS + 2 * PADR
TAB_PIX = B_IMG * ROWS

NP0 = 50000
NPP = 51200
NS0 = 10000
NSP = 10240
EP0, EPP = 800000, 808960
ES0, ESP = 320000, 327680
CHK = 128
CNT_N = 73728
CNT_E = 1212416


def _sc_mesh():
  return plsc.VectorSubcoreMesh(core_axis_name="c", subcore_axis_name="s")


_SC_PARAMS = pltpu.CompilerParams(use_tc_tiling_on_sc=False)


def _pix_gather(table, idx2d):
  CH, CBATCH = 64, 5
  epw = NPP // NW
  nb = epw // (CH * CBATCH)

  @functools.partial(
      pl.kernel,
      out_type=jax.ShapeDtypeStruct((NPP, 64), F32),
      mesh=_sc_mesh(),
      compiler_params=_SC_PARAMS,
      scratch_types=[
          pltpu.VMEM((CBATCH, CH), jnp.int32),
          pltpu.VMEM((CBATCH, CH, 64), F32),
          pltpu.SemaphoreType.DMA,
      ],
  )
  def k(tab_hbm, idx_hbm, out_hbm, idx_v, rows_v, sem):
    cid = lax.axis_index("c")
    sid = lax.axis_index("s")
    wid = sid * NC + cid
    base_chunk = wid * (epw // CH)

    def body(b, _):
      row0 = base_chunk + b * CBATCH
      pltpu.sync_copy(idx_hbm.at[wid * nb + b], idx_v)
      descs = [
          pltpu.async_copy(tab_hbm.at[idx_v.at[j]], rows_v.at[j], sem)
          for j in range(CBATCH)
      ]
      for j in range(CBATCH):
        descs[j].wait()
        pltpu.sync_copy(rows_v.at[j],
                        out_hbm.at[pl.ds((row0 + j) * CH, CH)])
      return 0

    lax.fori_loop(0, nb, body, 0)

  return k(table, idx2d)


def _edge_agg(table_flat, srcoff, dst2d, n_out, feat, nq, e_pad, cbatch, chk=CHK, dr=128):
  nqc = nq // NC
  ept = e_pad // NSUB
  nb = ept // (chk * cbatch)
  rpt = n_out // NSUB
  DR = dr
  nzc = rpt // DR

  @functools.partial(
      pl.kernel,
      out_type=jax.ShapeDtypeStruct((nq * n_out, feat), F32),
      mesh=_sc_mesh(),
      compiler_params=_SC_PARAMS,
      scratch_types=[
          pltpu.VMEM((2, cbatch, chk), jnp.int32),
          pltpu.VMEM((2, cbatch, chk), jnp.int32),
          pltpu.VMEM((cbatch, chk, feat), F32),
          pltpu.VMEM((DR, feat), F32),
          pltpu.VMEM((DR, feat), F32),
          pltpu.VMEM_SHARED((n_out, feat), F32),
          pltpu.SemaphoreType.DMA,
          pltpu.SemaphoreType.DMA,
          pltpu.SemaphoreType.DMA,
      ],
  )
  def k(tab_hbm, src_hbm, dst_hbm, zeros_hbm, out_hbm,
        sidx, didx, rows, zb, dump, acc, sem, isem, ssem):
    cid = lax.axis_index("c")
    sid = lax.axis_index("s")
    pltpu.sync_copy(zeros_hbm, zb)
    base_b = sid * nb

    for qq in range(nqc):
      q = cid * nqc + qq

      def zbody(i, _):
        pltpu.sync_copy(zb, acc.at[pl.ds(sid * rpt + i * DR, DR)])
        return 0
      lax.fori_loop(0, nzc, zbody, 0)
      plsc.subcore_barrier()

      pltpu.sync_copy(src_hbm.at[q, base_b], sidx.at[0])
      pltpu.sync_copy(dst_hbm.at[base_b], didx.at[0])

      def ebody(b, _):
        cur = b % 2
        nxt = (b + 1) % 2
        descs = [
            pltpu.async_copy(tab_hbm.at[sidx.at[cur, j]], rows.at[j], sem)
            for j in range(cbatch)
        ]

        @pl.when(b + 1 < nb)
        def _():
          pltpu.async_copy(src_hbm.at[q, base_b + b + 1],
                           sidx.at[nxt], isem)
          pltpu.async_copy(dst_hbm.at[base_b + b + 1],
                           didx.at[nxt], isem)

        sdescs = []
        for j in range(cbatch):
          descs[j].wait()
          sdescs.append(pltpu.async_copy(rows.at[j], acc.at[didx.at[cur, j]],
                                         ssem, add=True))
        for d in sdescs:
          d.wait()

        @pl.when(b + 1 < nb)
        def _():
          pltpu.make_async_copy(src_hbm.at[q, base_b], sidx.at[0],
                                isem).wait()
          pltpu.make_async_copy(dst_hbm.at[base_b], didx.at[0],
                                isem).wait()
        return 0
      lax.fori_loop(0, nb, ebody, 0)
      plsc.subcore_barrier()

      def dbody(i, _):
        r0 = sid * rpt + i * DR
        pltpu.sync_copy(acc.at[pl.ds(r0, DR)], dump)
        pltpu.sync_copy(dump, out_hbm.at[pl.ds(q * n_out + r0, DR)])
        return 0
      lax.fori_loop(0, nzc, dbody, 0)
      plsc.subcore_barrier()

  return k(table_flat, srcoff, dst2d, jnp.zeros((DR, feat), F32))


def _edge_agg_full(table, srcoff, dst2d, n_out, e_pad, cbatch, chk, dr=32):
  nb = e_pad // (NW * chk * cbatch)
  rpt = n_out // NSUB
  DR = dr
  nzc = rpt // DR

  @functools.partial(
      pl.kernel,
      out_type=jax.ShapeDtypeStruct((2 * n_out, 128), F32),
      mesh=_sc_mesh(),
      scratch_types=[
          pltpu.VMEM((2, cbatch, chk), jnp.int32),
          pltpu.VMEM((2, cbatch, chk), jnp.int32),
          pltpu.VMEM((cbatch, chk, 128), F32),
          pltpu.VMEM((DR, 128), F32),
          pltpu.VMEM((DR, 128), F32),
          pltpu.VMEM_SHARED((n_out, 128), F32),
          pltpu.SemaphoreType.DMA,
          pltpu.SemaphoreType.DMA,
          pltpu.SemaphoreType.DMA,
      ],
  )
  def k(tab_hbm, src_hbm, dst_hbm, zeros_hbm, out_hbm,
        sidx, didx, rows, zb, dump, acc, sem, isem, ssem):
    cid = lax.axis_index("c")
    sid = lax.axis_index("s")
    wid = sid * NC + cid
    pltpu.sync_copy(zeros_hbm, zb)
    base_b = wid * nb

    def zbody(i, _):
      pltpu.sync_copy(zb, acc.at[pl.ds(sid * rpt + i * DR, DR)])
      return 0
    lax.fori_loop(0, nzc, zbody, 0)
    plsc.subcore_barrier()

    pltpu.sync_copy(src_hbm.at[0, base_b], sidx.at[0])
    pltpu.sync_copy(dst_hbm.at[base_b], didx.at[0])

    def ebody(b, _):
      cur = b % 2
      nxt = (b + 1) % 2
      descs = [
          pltpu.async_copy(tab_hbm.at[sidx.at[cur, j]], rows.at[j], sem)
          for j in range(cbatch)
      ]

      @pl.when(b + 1 < nb)
      def _():
        pltpu.async_copy(src_hbm.at[0, base_b + b + 1], sidx.at[nxt], isem)
        pltpu.async_copy(dst_hbm.at[base_b + b + 1], didx.at[nxt], isem)

      sdescs = []
      for j in range(cbatch):
        descs[j].wait()
        sdescs.append(pltpu.async_copy(rows.at[j], acc.at[didx.at[cur, j]],
                                       ssem, add=True))
      for d in sdescs:
        d.wait()

      @pl.when(b + 1 < nb)
      def _():
        pltpu.make_async_copy(src_hbm.at[0, base_b], sidx.at[0], isem).wait()
        pltpu.make_async_copy(dst_hbm.at[base_b], didx.at[0], isem).wait()
      return 0
    lax.fori_loop(0, nb, ebody, 0)
    plsc.subcore_barrier()

    def dbody(i, _):
      r0 = sid * rpt + i * DR
      pltpu.sync_copy(acc.at[pl.ds(r0, DR)], dump)
      pltpu.sync_copy(dump, out_hbm.at[pl.ds(cid * n_out + r0, DR)])
      return 0
    lax.fori_loop(0, nzc, dbody, 0)

  return k(table, srcoff, dst2d, jnp.zeros((DR, 128), F32))


def _counts(dst2d, ones_h, zeros_h):
  CBATCH = 8
  epw = CNT_E // NW
  nb = epw // (CHK * CBATCH)
  rpt = CNT_N // NSUB
  DR = 512
  nzc = rpt // DR

  @functools.partial(
      pl.kernel,
      out_type=jax.ShapeDtypeStruct((2, CNT_N), F32),
      mesh=_sc_mesh(),
      compiler_params=_SC_PARAMS,
      scratch_types=[
          pltpu.VMEM((CBATCH, CHK), jnp.int32),
          pltpu.VMEM((CHK,), F32),
          pltpu.VMEM((DR,), F32),
          pltpu.VMEM((DR,), F32),
          pltpu.VMEM_SHARED((CNT_N,), F32),
      ],
  )
  def k(dst_hbm, ones_hbm, zeros_hbm, out_hbm, didx, ones_v, zb, dump, acc):
    cid = lax.axis_index("c")
    sid = lax.axis_index("s")
    wid = sid * NC + cid
    pltpu.sync_copy(ones_hbm, ones_v)
    pltpu.sync_copy(zeros_hbm, zb)

    def zbody(i, _):
      pltpu.sync_copy(zb, acc.at[pl.ds(sid * rpt + i * DR, DR)])
      return 0
    lax.fori_loop(0, nzc, zbody, 0)
    plsc.subcore_barrier()

    def ebody(b, _):
      pltpu.sync_copy(dst_hbm.at[wid * nb + b], didx)
      for j in range(CBATCH):
        pltpu.sync_copy(ones_v, acc.at[didx.at[j]], add=True)
      return 0
    lax.fori_loop(0, nb, ebody, 0)
    plsc.subcore_barrier()

    def dbody(i, _):
      r0 = sid * rpt + i * DR
      pltpu.sync_copy(acc.at[pl.ds(r0, DR)], dump)
      pltpu.sync_copy(dump, out_hbm.at[cid, pl.ds(r0, DR)])
      return 0
    lax.fori_loop(0, nzc, dbody, 0)

  return k(dst2d, ones_h, zeros_h)


def _conv_frontend(xpad, w1k, s1, t1, w2k, s2, t2, mask):
  def body(x_ref, w1_ref, s1_ref, t1_ref, w2_ref, s2_ref, t2_ref, m_ref,
           out_ref):
    x = x_ref[0]
    acc = jnp.zeros((ROWS, 64), F32)
    for k in range(9):
      o = PADR + (k // 3 - 1) * GRID + (k % 3 - 1)
      acc = acc + jnp.dot(x[o:o + ROWS, :], w1_ref[k],
                          preferred_element_type=F32)
    h = jnp.maximum(acc * s1_ref[...] + t1_ref[...], 0.0) * m_ref[...]
    zer = jnp.zeros((PADR, 64), F32)
    xp = jnp.concatenate([zer, h, zer], axis=0)
    acc2 = jnp.zeros((ROWS, 64), F32)
    for k in range(9):
      o = PADR + (k // 3 - 1) * GRID + (k % 3 - 1)
      acc2 = acc2 + jnp.dot(xp[o:o + ROWS, :], w2_ref[k],
                            preferred_element_type=F32)
    out_ref[0] = jnp.maximum(acc2 * s2_ref[...] + t2_ref[...], 0.0)

  def full(shp):
    return pl.BlockSpec(shp, lambda b: (0,) * len(shp))

  return pl.pallas_call(
      body,
      out_shape=jax.ShapeDtypeStruct((B_IMG, ROWS, 64), F32),
      grid=(B_IMG,),
      in_specs=[
          pl.BlockSpec((1, XROWS, 64), lambda b: (b, 0, 0)),
          full((9, 64, 64)), full((1, 64)), full((1, 64)),
          full((9, 64, 64)), full((1, 64)), full((1, 64)),
          full((ROWS, 1)),
      ],
      out_specs=pl.BlockSpec((1, ROWS, 64), lambda b: (b, 0, 0)),
  )(xpad, w1k, s1, t1, w2k, s2, t2, mask)


def _linear(x_parts, w, s, t, counts, n, blk, sin, win, sout, relu):
  fout = w.shape[1]
  wout = fout // sout
  use_cnt = counts is not None

  def body(*refs):
    if use_cnt:
      x_ref, w_ref, s_ref, t_ref, c_ref, out_ref = refs
    else:
      x_ref, w_ref, s_ref, t_ref, out_ref = refs
    y = jnp.zeros((blk, fout), F32)
    for p in range(sin):
      y = y + jnp.dot(x_ref[p], w_ref[p * win:(p + 1) * win, :],
                      preferred_element_type=F32)
    if use_cnt:
      c = jnp.maximum(c_ref[0] + c_ref[1], 1.0)
      y = y / c
    y = y * s_ref[...] + t_ref[...]
    if relu:
      y = jnp.maximum(y, 0.0)
    for q in range(sout):
      out_ref[q] = y[:, q * wout:(q + 1) * wout]

  in_specs = [
      pl.BlockSpec((sin, blk, win), lambda i: (0, i, 0)),
      pl.BlockSpec((sin * win, fout), lambda i: (0, 0)),
      pl.BlockSpec((1, fout), lambda i: (0, 0)),
      pl.BlockSpec((1, fout), lambda i: (0, 0)),
  ]
  args = [x_parts, w, s, t]
  if use_cnt:
    in_specs.append(pl.BlockSpec((2, blk, 1), lambda i: (0, i, 0)))
    args.append(counts)
  return pl.pallas_call(
      body,
      out_shape=jax.ShapeDtypeStruct((sout, n, wout), F32),
      grid=(n // blk,),
      in_specs=in_specs,
      out_specs=pl.BlockSpec((sout, blk, wout), lambda i: (0, i, 0)),
  )(*args)


def _readout(g_parts, ids, att_w, ro_w):
  blk = 2048
  nsteps = NSP // blk
  neg = float('-inf')

  def body(g_ref, id_ref, aw_ref, rw_ref, out_ref, hg):
    i = pl.program_id(0)

    @pl.when(i == 0)
    def _():
      hg[...] = jnp.full((16, 128), neg, F32)

    gg = g_ref[0]
    att = jnp.dot(gg, aw_ref[...], preferred_element_type=F32)
    sc = (1.0 / (1.0 + jnp.exp(-att)) + 1.0) * 0.5
    xx = gg * sc
    ids_b = id_ref[...]
    for gph in range(16):
      m = ids_b == gph
      c = jnp.max(jnp.where(m, xx, neg), axis=0, keepdims=True)
      hg[gph:gph + 1, :] = jnp.maximum(hg[gph:gph + 1, :], c)

    @pl.when(i == nsteps - 1)
    def _():
      out_ref[...] = jnp.dot(hg[...], rw_ref[...],
                             preferred_element_type=F32)

  return pl.pallas_call(
      body,
      out_shape=jax.ShapeDtypeStruct((16, 10), F32),
      grid=(nsteps,),
      in_specs=[
          pl.BlockSpec((1, blk, 128), lambda i: (0, i, 0)),
          pl.BlockSpec((blk, 1), lambda i: (i, 0)),
          pl.BlockSpec((128, 1), lambda i: (0, 0)),
          pl.BlockSpec((128, 10), lambda i: (0, 0)),
      ],
      out_specs=pl.BlockSpec((16, 10), lambda i: (0, 0)),
      scratch_shapes=[pltpu.VMEM((16, 128), F32)],
  )(g_parts, ids, att_w, ro_w)


def _fold_bn(g, b):
  s = g / jnp.sqrt(1.0 + EPS)
  return s[None, :], b[None, :]


def _fold_conv_bn(cb, g, b):
  s = g / jnp.sqrt(1.0 + EPS)
  return s[None, :], (cb * s + b)[None, :]


def _conv_w(w):
  wk = jnp.transpose(w, (2, 3, 1, 0)).reshape(9, w.shape[1], w.shape[0])
  if wk.shape[1] < 64:
    wk = jnp.pad(wk, ((0, 0), (0, 64 - wk.shape[1]), (0, 0)))
  return wk


def _pad_edges(src, dst, e_pad, n_tab, n_out, nq, cbatch, chk=CHK):
  e = src.shape[0]
  pad = e_pad - e
  ar = jnp.arange(pad, dtype=jnp.int32)
  src_p = jnp.concatenate([src.astype(jnp.int32), ar % 16])
  dst_p = jnp.concatenate([dst.astype(jnp.int32), n_out + (ar % 16)])
  offs = (jnp.arange(nq, dtype=jnp.int32) * n_tab)[:, None]
  nbatch = e_pad // (chk * cbatch)
  srcoff = (src_p[None, :] + offs).reshape(nq, nbatch, cbatch, chk)
  return srcoff, dst_p.reshape(nbatch, cbatch, chk)


def kernel(images, data_where, pixel_edge_index, pixel_batch, edge_index,
           graph_batch, params):
  p = params

  x = jnp.transpose(images, (0, 2, 3, 1))
  x = jnp.pad(x, ((0, 0), (1, 1), (1, 1), (0, 61)))
  x = x.reshape(B_IMG, ROWS, 64)
  x = jnp.pad(x, ((0, 0), (PADR, PADR), (0, 0)))

  rr = jnp.arange(ROWS)
  hh, ww = rr // GRID, rr % GRID
  interior = (hh >= 1) & (hh <= 64) & (ww >= 1) & (ww <= 64)
  mask = interior.astype(F32)[:, None]

  s1c, t1c = _fold_conv_bn(p['conv1_b'], p['bn1_g'], p['bn1_b'])
  s2c, t2c = _fold_conv_bn(p['conv2_b'], p['bn2_g'], p['bn2_b'])
  feat = _conv_frontend(x, _conv_w(p['conv1_w']), s1c, t1c,
                        _conv_w(p['conv2_w']), s2c, t2c, mask)
  table0 = feat.reshape(TAB_PIX, 64)

  dw = data_where.astype(jnp.int32)
  pidx = dw[:, 0] * ROWS + (dw[:, 1] + 1) * GRID + (dw[:, 2] + 1)
  pidx = jnp.concatenate(
      [pidx, jnp.zeros((NPP - NP0,), jnp.int32)]).reshape(160, 5, 64)
  pix = _pix_gather(table0, pidx)

  src_px = pixel_edge_index[0].astype(jnp.int32)
  dst_px = pixel_edge_index[1].astype(jnp.int32)
  src_sp = edge_index[0].astype(jnp.int32)
  dst_sp = edge_index[1].astype(jnp.int32)
  pb = pixel_batch.astype(jnp.int32)

  cnt_dummy = 71680
  parts = [dst_px, jnp.full((EPP - EP0,), cnt_dummy, jnp.int32),
           NPP + pb, jnp.full((NPP - NP0,), cnt_dummy, jnp.int32),
           NPP + NSP + dst_sp, jnp.full((ESP - ES0,), cnt_dummy, jnp.int32)]
  all_dst = jnp.concatenate(parts)
  all_dst = jnp.concatenate(
      [all_dst, jnp.full((CNT_E - all_dst.shape[0],), cnt_dummy, jnp.int32)])
  counts2 = _counts(all_dst.reshape(CNT_E // (CHK * 8), 8, CHK),
                    jnp.ones((CHK,), F32), jnp.zeros((512,), F32))
  cnt_px = counts2[:, :NPP].reshape(2, NPP, 1)
  cnt_pool = counts2[:, NPP:NPP + NSP].reshape(2, NSP, 1)
  cnt_sp = counts2[:, NPP + NSP:NPP + 2 * NSP].reshape(2, NSP, 1)

  s_emb, t_emb = jnp.ones((1, 64), F32), p['s1_emb_b'][None, :]
  h0 = _linear(pix.reshape(1, NPP, 64), p['s1_emb_w'], s_emb, t_emb,
               None, NPP, 3200, 1, 64, 2, False)

  so_px2, dst2_px = _pad_edges(src_px, dst_px, EPP, NPP, NP0, 2, 5, 128)
  agg0 = _edge_agg(h0.reshape(2 * NPP, 32), so_px2, dst2_px,
                   NPP, 32, 2, EPP, 5, 128, 64)
  bs, bt = _fold_bn(p['s1_bn0_g'], p['s1_bn0_b'])
  bt = p['s1_l0_b'][None, :] * bs + bt
  h1 = _linear(agg0.reshape(2, NPP, 32), p['s1_l0_w'], bs, bt,
               cnt_px, NPP, 3200, 2, 32, 4, True)

  so_px4, _ = _pad_edges(src_px, dst_px, EPP, NPP, NP0, 4, 5, 128)
  agg1 = _edge_agg(h1.reshape(4 * NPP, 32), so_px4, dst2_px,
                   NPP, 32, 4, EPP, 5, 128, 64)
  bs, bt = _fold_bn(p['s1_bn1_g'], p['s1_bn1_b'])
  bt = p['s1_l1_b'][None, :] * bs + bt
  h2 = _linear(agg1.reshape(4, NPP, 32), p['s1_l1_w'], bs, bt,
               cnt_px, NPP, 3200, 4, 32, 1, True)

  pool_src = jnp.arange(NPP, dtype=jnp.int32)
  pool_dst = jnp.concatenate(
      [pb, NS0 + (jnp.arange(NPP - NP0, dtype=jnp.int32) % 16)])
  so_pool, dst2_pool = _pad_edges(pool_src, pool_dst, NPP, NPP, NS0, 1, 4, 50)
  gcn1 = _edge_agg_full(h2.reshape(NPP, 128), so_pool, dst2_pool,
                        NSP, NPP, 4, 50)

  s_emb2, t_emb2 = jnp.ones((1, 128), F32), p['s2_emb_b'][None, :]
  w_emb2 = jnp.concatenate([p['s2_emb_w'], p['s2_emb_w']], axis=0)
  g = _linear(gcn1.reshape(2, NSP, 128), w_emb2, s_emb2, t_emb2,
              cnt_pool, NSP, 2048, 2, 128, 1, False)

  so_sp, dst2_sp = _pad_edges(src_sp, dst_sp, ESP, NSP, NS0, 1, 5, 64)
  for i in range(4):
    agg = _edge_agg_full(g.reshape(NSP, 128), so_sp, dst2_sp,
                         NSP, ESP, 5, 64, 16)
    bs, bt = _fold_bn(p['s2_bn%d_g' % i], p['s2_bn%d_b' % i])
    bt = p['s2_l%d_b' % i][None, :] * bs + bt
    wl = jnp.concatenate([p['s2_l%d_w' % i], p['s2_l%d_w' % i]], axis=0)
    g = _linear(agg.reshape(2, NSP, 128), wl, bs, bt,
                cnt_sp, NSP, 2048, 2, 128, 1, True)

  gb = jnp.concatenate(
      [graph_batch.astype(jnp.int32),
       jnp.full((NSP - NS0,), 16, jnp.int32)]).reshape(NSP, 1)
  return _readout(g.reshape(1, NSP, 128), gb, p['att_w'], p['ro_w'])


# speedup vs baseline: 1.1050x; 1.0235x over previous
import functools

import jax
import jax.numpy as jnp
from jax import lax
from jax.experimental import pallas as pl
from jax.experimental.pallas import tpu as pltpu
from jax.experimental.pallas import tpu_sc as plsc

EPS = 1e-5
F32 = jnp.float32

NC, NSUB = 2, 16
NW = NC * NSUB

B_IMG = 16
GRID = 66
ROWS = GRID * GRID
PADR = 67
XROWS = ROWS + 2 * PADR
TAB_PIX = B_IMG * ROWS

NP0 = 50000
NPP = 51200
NS0 = 10000
NSP = 10240
EP0, EPP = 800000, 802816
ES0, ESP = 320000, 327680
CHK = 128
CNT_N = 73728
CNT_E = 1212416


def _sc_mesh():
  return plsc.VectorSubcoreMesh(core_axis_name="c", subcore_axis_name="s")


_SC_PARAMS = pltpu.CompilerParams(use_tc_tiling_on_sc=False)


def _pix_gather(table, idx2d):
  CH, CBATCH = 64, 5
  epw = NPP // NW
  nb = epw // (CH * CBATCH)

  @functools.partial(
      pl.kernel,
      out_type=jax.ShapeDtypeStruct((NPP, 64), F32),
      mesh=_sc_mesh(),
      compiler_params=_SC_PARAMS,
      scratch_types=[
          pltpu.VMEM((CBATCH, CH), jnp.int32),
          pltpu.VMEM((CBATCH, CH, 64), F32),
          pltpu.SemaphoreType.DMA,
      ],
  )
  def k(tab_hbm, idx_hbm, out_hbm, idx_v, rows_v, sem):
    cid = lax.axis_index("c")
    sid = lax.axis_index("s")
    wid = sid * NC + cid
    base_chunk = wid * (epw // CH)

    def body(b, _):
      row0 = base_chunk + b * CBATCH
      pltpu.sync_copy(idx_hbm.at[wid * nb + b], idx_v)
      descs = [
          pltpu.async_copy(tab_hbm.at[idx_v.at[j]], rows_v.at[j], sem)
          for j in range(CBATCH)
      ]
      for j in range(CBATCH):
        descs[j].wait()
        pltpu.sync_copy(rows_v.at[j],
                        out_hbm.at[pl.ds((row0 + j) * CH, CH)])
      return 0

    lax.fori_loop(0, nb, body, 0)

  return k(table, idx2d)


def _edge_agg(table_flat, srcoff, dst2d, n_out, feat, nq, e_pad, cbatch, chk=CHK, dr=128):
  nqc = nq // NC
  ept = e_pad // NSUB
  nb = ept // (chk * cbatch)
  rpt = n_out // NSUB
  DR = dr
  nzc = rpt // DR

  @functools.partial(
      pl.kernel,
      out_type=jax.ShapeDtypeStruct((nq * n_out, feat), F32),
      mesh=_sc_mesh(),
      compiler_params=_SC_PARAMS,
      scratch_types=[
          pltpu.VMEM((2, cbatch, chk), jnp.int32),
          pltpu.VMEM((2, cbatch, chk), jnp.int32),
          pltpu.VMEM((cbatch, chk, feat), F32),
          pltpu.VMEM((DR, feat), F32),
          pltpu.VMEM((DR, feat), F32),
          pltpu.VMEM_SHARED((n_out, feat), F32),
          pltpu.SemaphoreType.DMA,
          pltpu.SemaphoreType.DMA,
          pltpu.SemaphoreType.DMA,
      ],
  )
  def k(tab_hbm, src_hbm, dst_hbm, zeros_hbm, out_hbm,
        sidx, didx, rows, zb, dump, acc, sem, isem, ssem):
    cid = lax.axis_index("c")
    sid = lax.axis_index("s")
    pltpu.sync_copy(zeros_hbm, zb)
    base_b = sid * nb

    for qq in range(nqc):
      q = cid * nqc + qq

      def zbody(i, _):
        pltpu.sync_copy(zb, acc.at[pl.ds(sid * rpt + i * DR, DR)])
        return 0
      lax.fori_loop(0, nzc, zbody, 0)
      plsc.subcore_barrier()

      pltpu.sync_copy(src_hbm.at[q, base_b], sidx.at[0])
      pltpu.sync_copy(dst_hbm.at[base_b], didx.at[0])

      def ebody(b, _):
        cur = b % 2
        nxt = (b + 1) % 2
        descs = [
            pltpu.async_copy(tab_hbm.at[sidx.at[cur, j]], rows.at[j], sem)
            for j in range(cbatch)
        ]

        @pl.when(b + 1 < nb)
        def _():
          pltpu.async_copy(src_hbm.at[q, base_b + b + 1],
                           sidx.at[nxt], isem)
          pltpu.async_copy(dst_hbm.at[base_b + b + 1],
                           didx.at[nxt], isem)

        sdescs = []
        for j in range(cbatch):
          descs[j].wait()
          sdescs.append(pltpu.async_copy(rows.at[j], acc.at[didx.at[cur, j]],
                                         ssem, add=True))
        for d in sdescs:
          d.wait()

        @pl.when(b + 1 < nb)
        def _():
          pltpu.make_async_copy(src_hbm.at[q, base_b], sidx.at[0],
                                isem).wait()
          pltpu.make_async_copy(dst_hbm.at[base_b], didx.at[0],
                                isem).wait()
        return 0
      lax.fori_loop(0, nb, ebody, 0)
      plsc.subcore_barrier()

      def dbody(i, _):
        r0 = sid * rpt + i * DR
        pltpu.sync_copy(acc.at[pl.ds(r0, DR)], dump)
        pltpu.sync_copy(dump, out_hbm.at[pl.ds(q * n_out + r0, DR)])
        return 0
      lax.fori_loop(0, nzc, dbody, 0)
      plsc.subcore_barrier()

  return k(table_flat, srcoff, dst2d, jnp.zeros((DR, feat), F32))


def _edge_agg_full(table, srcoff, dst2d, n_out, e_pad, cbatch, chk, dr=32):
  nb = e_pad // (NW * chk * cbatch)
  rpt = n_out // NSUB
  DR = dr
  nzc = rpt // DR

  @functools.partial(
      pl.kernel,
      out_type=jax.ShapeDtypeStruct((2 * n_out, 128), F32),
      mesh=_sc_mesh(),
      scratch_types=[
          pltpu.VMEM((2, cbatch, chk), jnp.int32),
          pltpu.VMEM((2, cbatch, chk), jnp.int32),
          pltpu.VMEM((cbatch, chk, 128), F32),
          pltpu.VMEM((DR, 128), F32),
          pltpu.VMEM((DR, 128), F32),
          pltpu.VMEM_SHARED((n_out, 128), F32),
          pltpu.SemaphoreType.DMA,
          pltpu.SemaphoreType.DMA,
          pltpu.SemaphoreType.DMA,
      ],
  )
  def k(tab_hbm, src_hbm, dst_hbm, zeros_hbm, out_hbm,
        sidx, didx, rows, zb, dump, acc, sem, isem, ssem):
    cid = lax.axis_index("c")
    sid = lax.axis_index("s")
    wid = sid * NC + cid
    pltpu.sync_copy(zeros_hbm, zb)
    base_b = wid * nb

    def zbody(i, _):
      pltpu.sync_copy(zb, acc.at[pl.ds(sid * rpt + i * DR, DR)])
      return 0
    lax.fori_loop(0, nzc, zbody, 0)
    plsc.subcore_barrier()

    pltpu.sync_copy(src_hbm.at[0, base_b], sidx.at[0])
    pltpu.sync_copy(dst_hbm.at[base_b], didx.at[0])

    def ebody(b, _):
      cur = b % 2
      nxt = (b + 1) % 2
      descs = [
          pltpu.async_copy(tab_hbm.at[sidx.at[cur, j]], rows.at[j], sem)
          for j in range(cbatch)
      ]

      @pl.when(b + 1 < nb)
      def _():
        pltpu.async_copy(src_hbm.at[0, base_b + b + 1], sidx.at[nxt], isem)
        pltpu.async_copy(dst_hbm.at[base_b + b + 1], didx.at[nxt], isem)

      sdescs = []
      for j in range(cbatch):
        descs[j].wait()
        sdescs.append(pltpu.async_copy(rows.at[j], acc.at[didx.at[cur, j]],
                                       ssem, add=True))
      for d in sdescs:
        d.wait()

      @pl.when(b + 1 < nb)
      def _():
        pltpu.make_async_copy(src_hbm.at[0, base_b], sidx.at[0], isem).wait()
        pltpu.make_async_copy(dst_hbm.at[base_b], didx.at[0], isem).wait()
      return 0
    lax.fori_loop(0, nb, ebody, 0)
    plsc.subcore_barrier()

    def dbody(i, _):
      r0 = sid * rpt + i * DR
      pltpu.sync_copy(acc.at[pl.ds(r0, DR)], dump)
      pltpu.sync_copy(dump, out_hbm.at[pl.ds(cid * n_out + r0, DR)])
      return 0
    lax.fori_loop(0, nzc, dbody, 0)

  return k(table, srcoff, dst2d, jnp.zeros((DR, 128), F32))


def _counts(dst2d, ones_h, zeros_h):
  CBATCH = 8
  epw = CNT_E // NW
  nb = epw // (CHK * CBATCH)
  rpt = CNT_N // NSUB
  DR = 512
  nzc = rpt // DR

  @functools.partial(
      pl.kernel,
      out_type=jax.ShapeDtypeStruct((2, CNT_N), F32),
      mesh=_sc_mesh(),
      compiler_params=_SC_PARAMS,
      scratch_types=[
          pltpu.VMEM((CBATCH, CHK), jnp.int32),
          pltpu.VMEM((CHK,), F32),
          pltpu.VMEM((DR,), F32),
          pltpu.VMEM((DR,), F32),
          pltpu.VMEM_SHARED((CNT_N,), F32),
      ],
  )
  def k(dst_hbm, ones_hbm, zeros_hbm, out_hbm, didx, ones_v, zb, dump, acc):
    cid = lax.axis_index("c")
    sid = lax.axis_index("s")
    wid = sid * NC + cid
    pltpu.sync_copy(ones_hbm, ones_v)
    pltpu.sync_copy(zeros_hbm, zb)

    def zbody(i, _):
      pltpu.sync_copy(zb, acc.at[pl.ds(sid * rpt + i * DR, DR)])
      return 0
    lax.fori_loop(0, nzc, zbody, 0)
    plsc.subcore_barrier()

    def ebody(b, _):
      pltpu.sync_copy(dst_hbm.at[wid * nb + b], didx)
      for j in range(CBATCH):
        pltpu.sync_copy(ones_v, acc.at[didx.at[j]], add=True)
      return 0
    lax.fori_loop(0, nb, ebody, 0)
    plsc.subcore_barrier()

    def dbody(i, _):
      r0 = sid * rpt + i * DR
      pltpu.sync_copy(acc.at[pl.ds(r0, DR)], dump)
      pltpu.sync_copy(dump, out_hbm.at[cid, pl.ds(r0, DR)])
      return 0
    lax.fori_loop(0, nzc, dbody, 0)

  return k(dst2d, ones_h, zeros_h)


def _conv_frontend(xpad, w1k, s1, t1, w2k, s2, t2, mask):
  def body(x_ref, w1_ref, s1_ref, t1_ref, w2_ref, s2_ref, t2_ref, m_ref,
           out_ref):
    x = x_ref[0]
    acc = jnp.zeros((ROWS, 64), F32)
    for k in range(9):
      o = PADR + (k // 3 - 1) * GRID + (k % 3 - 1)
      acc = acc + jnp.dot(x[o:o + ROWS, :], w1_ref[k],
                          preferred_element_type=F32)
    h = jnp.maximum(acc * s1_ref[...] + t1_ref[...], 0.0) * m_ref[...]
    zer = jnp.zeros((PADR, 64), F32)
    xp = jnp.concatenate([zer, h, zer], axis=0)
    acc2 = jnp.zeros((ROWS, 64), F32)
    for k in range(9):
      o = PADR + (k // 3 - 1) * GRID + (k % 3 - 1)
      acc2 = acc2 + jnp.dot(xp[o:o + ROWS, :], w2_ref[k],
                            preferred_element_type=F32)
    out_ref[0] = jnp.maximum(acc2 * s2_ref[...] + t2_ref[...], 0.0)

  def full(shp):
    return pl.BlockSpec(shp, lambda b: (0,) * len(shp))

  return pl.pallas_call(
      body,
      out_shape=jax.ShapeDtypeStruct((B_IMG, ROWS, 64), F32),
      grid=(B_IMG,),
      in_specs=[
          pl.BlockSpec((1, XROWS, 64), lambda b: (b, 0, 0)),
          full((9, 64, 64)), full((1, 64)), full((1, 64)),
          full((9, 64, 64)), full((1, 64)), full((1, 64)),
          full((ROWS, 1)),
      ],
      out_specs=pl.BlockSpec((1, ROWS, 64), lambda b: (b, 0, 0)),
  )(xpad, w1k, s1, t1, w2k, s2, t2, mask)


def _linear(x_parts, w, s, t, counts, n, blk, sin, win, sout, relu):
  fout = w.shape[1]
  wout = fout // sout
  use_cnt = counts is not None

  def body(*refs):
    if use_cnt:
      x_ref, w_ref, s_ref, t_ref, c_ref, out_ref = refs
    else:
      x_ref, w_ref, s_ref, t_ref, out_ref = refs
    y = jnp.zeros((blk, fout), F32)
    for p in range(sin):
      y = y + jnp.dot(x_ref[p], w_ref[p * win:(p + 1) * win, :],
                      preferred_element_type=F32)
    if use_cnt:
      c = jnp.maximum(c_ref[0] + c_ref[1], 1.0)
      y = y / c
    y = y * s_ref[...] + t_ref[...]
    if relu:
      y = jnp.maximum(y, 0.0)
    for q in range(sout):
      out_ref[q] = y[:, q * wout:(q + 1) * wout]

  in_specs = [
      pl.BlockSpec((sin, blk, win), lambda i: (0, i, 0)),
      pl.BlockSpec((sin * win, fout), lambda i: (0, 0)),
      pl.BlockSpec((1, fout), lambda i: (0, 0)),
      pl.BlockSpec((1, fout), lambda i: (0, 0)),
  ]
  args = [x_parts, w, s, t]
  if use_cnt:
    in_specs.append(pl.BlockSpec((2, blk, 1), lambda i: (0, i, 0)))
    args.append(counts)
  return pl.pallas_call(
      body,
      out_shape=jax.ShapeDtypeStruct((sout, n, wout), F32),
      grid=(n // blk,),
      in_specs=in_specs,
      out_specs=pl.BlockSpec((sout, blk, wout), lambda i: (0, i, 0)),
  )(*args)


def _readout(g_parts, ids, att_w, ro_w):
  blk = 2048
  nsteps = NSP // blk
  neg = float('-inf')

  def body(g_ref, id_ref, aw_ref, rw_ref, out_ref, hg):
    i = pl.program_id(0)

    @pl.when(i == 0)
    def _():
      hg[...] = jnp.full((16, 128), neg, F32)

    gg = g_ref[0]
    att = jnp.dot(gg, aw_ref[...], preferred_element_type=F32)
    sc = (1.0 / (1.0 + jnp.exp(-att)) + 1.0) * 0.5
    xx = gg * sc
    ids_b = id_ref[...]
    for gph in range(16):
      m = ids_b == gph
      c = jnp.max(jnp.where(m, xx, neg), axis=0, keepdims=True)
      hg[gph:gph + 1, :] = jnp.maximum(hg[gph:gph + 1, :], c)

    @pl.when(i == nsteps - 1)
    def _():
      out_ref[...] = jnp.dot(hg[...], rw_ref[...],
                             preferred_element_type=F32)

  return pl.pallas_call(
      body,
      out_shape=jax.ShapeDtypeStruct((16, 10), F32),
      grid=(nsteps,),
      in_specs=[
          pl.BlockSpec((1, blk, 128), lambda i: (0, i, 0)),
          pl.BlockSpec((blk, 1), lambda i: (i, 0)),
          pl.BlockSpec((128, 1), lambda i: (0, 0)),
          pl.BlockSpec((128, 10), lambda i: (0, 0)),
      ],
      out_specs=pl.BlockSpec((16, 10), lambda i: (0, 0)),
      scratch_shapes=[pltpu.VMEM((16, 128), F32)],
  )(g_parts, ids, att_w, ro_w)


def _fold_bn(g, b):
  s = g / jnp.sqrt(1.0 + EPS)
  return s[None, :], b[None, :]


def _fold_conv_bn(cb, g, b):
  s = g / jnp.sqrt(1.0 + EPS)
  return s[None, :], (cb * s + b)[None, :]


def _conv_w(w):
  wk = jnp.transpose(w, (2, 3, 1, 0)).reshape(9, w.shape[1], w.shape[0])
  if wk.shape[1] < 64:
    wk = jnp.pad(wk, ((0, 0), (0, 64 - wk.shape[1]), (0, 0)))
  return wk


def _pad_edges(src, dst, e_pad, n_tab, n_out, nq, cbatch, chk=CHK):
  e = src.shape[0]
  pad = e_pad - e
  ar = jnp.arange(pad, dtype=jnp.int32)
  src_p = jnp.concatenate([src.astype(jnp.int32), ar % 16])
  dst_p = jnp.concatenate([dst.astype(jnp.int32), n_out + (ar % 16)])
  offs = (jnp.arange(nq, dtype=jnp.int32) * n_tab)[:, None]
  nbatch = e_pad // (chk * cbatch)
  srcoff = (src_p[None, :] + offs).reshape(nq, nbatch, cbatch, chk)
  return srcoff, dst_p.reshape(nbatch, cbatch, chk)


def kernel(images, data_where, pixel_edge_index, pixel_batch, edge_index,
           graph_batch, params):
  p = params

  x = jnp.transpose(images, (0, 2, 3, 1))
  x = jnp.pad(x, ((0, 0), (1, 1), (1, 1), (0, 61)))
  x = x.reshape(B_IMG, ROWS, 64)
  x = jnp.pad(x, ((0, 0), (PADR, PADR), (0, 0)))

  rr = jnp.arange(ROWS)
  hh, ww = rr // GRID, rr % GRID
  interior = (hh >= 1) & (hh <= 64) & (ww >= 1) & (ww <= 64)
  mask = interior.astype(F32)[:, None]

  s1c, t1c = _fold_conv_bn(p['conv1_b'], p['bn1_g'], p['bn1_b'])
  s2c, t2c = _fold_conv_bn(p['conv2_b'], p['bn2_g'], p['bn2_b'])
  feat = _conv_frontend(x, _conv_w(p['conv1_w']), s1c, t1c,
                        _conv_w(p['conv2_w']), s2c, t2c, mask)
  table0 = feat.reshape(TAB_PIX, 64)

  dw = data_where.astype(jnp.int32)
  pidx = dw[:, 0] * ROWS + (dw[:, 1] + 1) * GRID + (dw[:, 2] + 1)
  pidx = jnp.concatenate(
      [pidx, jnp.zeros((NPP - NP0,), jnp.int32)]).reshape(160, 5, 64)
  pix = _pix_gather(table0, pidx)

  src_px = pixel_edge_index[0].astype(jnp.int32)
  dst_px = pixel_edge_index[1].astype(jnp.int32)
  src_sp = edge_index[0].astype(jnp.int32)
  dst_sp = edge_index[1].astype(jnp.int32)
  pb = pixel_batch.astype(jnp.int32)

  cnt_dummy = 71680
  parts = [dst_px, jnp.full((EPP - EP0,), cnt_dummy, jnp.int32),
           NPP + pb, jnp.full((NPP - NP0,), cnt_dummy, jnp.int32),
           NPP + NSP + dst_sp, jnp.full((ESP - ES0,), cnt_dummy, jnp.int32)]
  all_dst = jnp.concatenate(parts)
  all_dst = jnp.concatenate(
      [all_dst, jnp.full((CNT_E - all_dst.shape[0],), cnt_dummy, jnp.int32)])
  counts2 = _counts(all_dst.reshape(CNT_E // (CHK * 8), 8, CHK),
                    jnp.ones((CHK,), F32), jnp.zeros((512,), F32))
  cnt_px = counts2[:, :NPP].reshape(2, NPP, 1)
  cnt_pool = counts2[:, NPP:NPP + NSP].reshape(2, NSP, 1)
  cnt_sp = counts2[:, NPP + NSP:NPP + 2 * NSP].reshape(2, NSP, 1)

  s_emb, t_emb = jnp.ones((1, 64), F32), p['s1_emb_b'][None, :]
  h0 = _linear(pix.reshape(1, NPP, 64), p['s1_emb_w'], s_emb, t_emb,
               None, NPP, 3200, 1, 64, 2, False)

  so_px2, dst2_px = _pad_edges(src_px, dst_px, EPP, NPP, NP0, 2, 4, 128)
  agg0 = _edge_agg(h0.reshape(2 * NPP, 32), so_px2, dst2_px,
                   NPP, 32, 2, EPP, 4, 128, 64)
  bs, bt = _fold_bn(p['s1_bn0_g'], p['s1_bn0_b'])
  bt = p['s1_l0_b'][None, :] * bs + bt
  h1 = _linear(agg0.reshape(2, NPP, 32), p['s1_l0_w'], bs, bt,
               cnt_px, NPP, 3200, 2, 32, 4, True)

  so_px4, _ = _pad_edges(src_px, dst_px, EPP, NPP, NP0, 4, 4, 128)
  agg1 = _edge_agg(h1.reshape(4 * NPP, 32), so_px4, dst2_px,
                   NPP, 32, 4, EPP, 4, 128, 64)
  bs, bt = _fold_bn(p['s1_bn1_g'], p['s1_bn1_b'])
  bt = p['s1_l1_b'][None, :] * bs + bt
  h2 = _linear(agg1.reshape(4, NPP, 32), p['s1_l1_w'], bs, bt,
               cnt_px, NPP, 3200, 4, 32, 1, True)

  pool_src = jnp.arange(NPP, dtype=jnp.int32)
  pool_dst = jnp.concatenate(
      [pb, NS0 + (jnp.arange(NPP - NP0, dtype=jnp.int32) % 16)])
  so_pool, dst2_pool = _pad_edges(pool_src, pool_dst, NPP, NPP, NS0, 1, 4, 50)
  gcn1 = _edge_agg_full(h2.reshape(NPP, 128), so_pool, dst2_pool,
                        NSP, NPP, 4, 50)

  s_emb2, t_emb2 = jnp.ones((1, 128), F32), p['s2_emb_b'][None, :]
  w_emb2 = jnp.concatenate([p['s2_emb_w'], p['s2_emb_w']], axis=0)
  g = _linear(gcn1.reshape(2, NSP, 128), w_emb2, s_emb2, t_emb2,
              cnt_pool, NSP, 2048, 2, 128, 1, False)

  so_sp, dst2_sp = _pad_edges(src_sp, dst_sp, ESP, NSP, NS0, 1, 4, 64)
  for i in range(4):
    agg = _edge_agg_full(g.reshape(NSP, 128), so_sp, dst2_sp,
                         NSP, ESP, 4, 64)
    bs, bt = _fold_bn(p['s2_bn%d_g' % i], p['s2_bn%d_b' % i])
    bt = p['s2_l%d_b' % i][None, :] * bs + bt
    wl = jnp.concatenate([p['s2_l%d_w' % i], p['s2_l%d_w' % i]], axis=0)
    g = _linear(agg.reshape(2, NSP, 128), wl, bs, bt,
                cnt_sp, NSP, 2048, 2, 128, 1, True)

  gb = jnp.concatenate(
      [graph_batch.astype(jnp.int32),
       jnp.full((NSP - NS0,), 16, jnp.int32)]).reshape(NSP, 1)
  return _readout(g.reshape(1, NSP, 128), gb, p['att_w'], p['ro_w'])
